# Initial kernel scaffold; baseline (speedup 1.0000x reference)
import jax
import jax.numpy as jnp
from jax.experimental import pallas as pl


def kernel(x, edge_index, W1, b1, W2, b2):
    raise NotImplementedError("write your pallas kernel here")


# speedup vs baseline: 3.1845x; 3.1845x over previous
import functools

import jax
import jax.numpy as jnp
from jax import lax
from jax.experimental import pallas as pl
from jax.experimental.pallas import tpu as pltpu
from jax.experimental.pallas import tpu_sc as plsc

N = 10000
D = 128
E = 320000

NUM_CORES = 2
NUM_SUBCORES = 16
NUM_WORKERS = NUM_CORES * NUM_SUBCORES

CHUNK = 128
N_PAD = 10240
E_PAD = 327680
N_CHUNKS = E_PAD // CHUNK
CHUNKS_PER_W = N_CHUNKS // NUM_WORKERS
ROWS_PER_TILE = N_PAD // NUM_SUBCORES
LANES = 16


def _sc_aggregate(table, src2d, dst2d, zrows, with_deg):
    mesh = plsc.VectorSubcoreMesh(core_axis_name="c", subcore_axis_name="s")

    out_type = [jax.ShapeDtypeStruct((NUM_CORES, N_PAD, D), jnp.float32)]
    scratch = [
        pltpu.VMEM((CHUNK,), jnp.int32),
        pltpu.VMEM((CHUNK,), jnp.int32),
        pltpu.VMEM((CHUNK, D), jnp.float32),
        pltpu.VMEM_SHARED((N_PAD, D), jnp.float32),
        pltpu.SemaphoreType.DMA,
    ]
    if with_deg:
        out_type.append(jax.ShapeDtypeStruct((NUM_CORES, N_PAD), jnp.float32))
        scratch += [
            pltpu.VMEM((N_PAD,), jnp.float32),
            pltpu.VMEM((ROWS_PER_TILE,), jnp.float32),
            pltpu.VMEM((ROWS_PER_TILE,), jnp.float32),
            pltpu.VMEM_SHARED((NUM_SUBCORES, N_PAD), jnp.float32),
        ]

    @functools.partial(
        pl.kernel, mesh=mesh,
        compiler_params=pltpu.CompilerParams(needs_layout_passes=False),
        out_type=out_type, scratch_types=scratch)
    def k(table_hbm, src_hbm, dst_hbm, zrows_hbm, agg_out, *rest):
        if with_deg:
            (deg_out, idx_s, idx_d, rows, agg_sh, sem,
             deg_v, acc_v, tmp_v, stage) = rest
        else:
            idx_s, idx_d, rows, agg_sh, sem = rest

        cid = lax.axis_index("c")
        sid = lax.axis_index("s")
        wid = cid * NUM_SUBCORES + sid
        row0 = sid * ROWS_PER_TILE

        pltpu.sync_copy(zrows_hbm, agg_sh.at[pl.ds(row0, ROWS_PER_TILE)])
        if with_deg:
            def zb(j, c):
                deg_v[pl.ds(j * LANES, LANES)] = jnp.zeros((LANES,),
                                                           jnp.float32)
                return c
            lax.fori_loop(0, N_PAD // LANES, zb, 0)
        plsc.subcore_barrier()

        chunk0 = wid * CHUNKS_PER_W

        def chunk_body(c, carry):
            ci = chunk0 + c
            pltpu.sync_copy(src_hbm.at[ci], idx_s)
            pltpu.sync_copy(dst_hbm.at[ci], idx_d)
            pltpu.async_copy(table_hbm.at[idx_s], rows, sem).wait()
            pltpu.sync_copy(rows, agg_sh.at[idx_d], add=True)
            if with_deg:
                def hb(j, c2):
                    iv = idx_d[pl.ds(j * LANES, LANES)]
                    plsc.addupdate_scatter(
                        deg_v, [iv], jnp.ones((LANES,), jnp.float32))
                    return c2
                lax.fori_loop(0, CHUNK // LANES, hb, 0)
            return carry

        lax.fori_loop(0, CHUNKS_PER_W, chunk_body, 0)

        if with_deg:
            pltpu.sync_copy(deg_v, stage.at[sid])
        plsc.subcore_barrier()

        pltpu.sync_copy(agg_sh.at[pl.ds(row0, ROWS_PER_TILE)],
                        agg_out.at[cid, pl.ds(row0, ROWS_PER_TILE)])

        if with_deg:
            def zb2(j, c):
                acc_v[pl.ds(j * LANES, LANES)] = jnp.zeros((LANES,),
                                                           jnp.float32)
                return c
            lax.fori_loop(0, ROWS_PER_TILE // LANES, zb2, 0)

            def rb(t, c):
                pltpu.sync_copy(stage.at[t, pl.ds(row0, ROWS_PER_TILE)],
                                tmp_v)

                def ab(j, c2):
                    s = pl.ds(j * LANES, LANES)
                    acc_v[s] = acc_v[s] + tmp_v[s]
                    return c2
                lax.fori_loop(0, ROWS_PER_TILE // LANES, ab, 0)
                return c
            lax.fori_loop(0, NUM_SUBCORES, rb, 0)
            pltpu.sync_copy(acc_v, deg_out.at[cid, pl.ds(row0,
                                                         ROWS_PER_TILE)])

    res = k(table, src2d, dst2d, zrows)
    if not isinstance(res, (list, tuple)):
        res = (res,)
    return res


BLK = 1024
GRID = N_PAD // BLK


def _tc_layer1(agg0, agg1, deg0, deg1, W, b2d):
    def body(a0, a1, d0, d1, w, b, h_ref, dinv_ref):
        deg = d0[...] + d1[...]
        dinv = 1.0 / jnp.maximum(deg, 1.0)
        a = (a0[...] + a1[...]) * dinv
        h = jnp.dot(a, w[...], preferred_element_type=jnp.float32) + b[...]
        h_ref[...] = jnp.maximum(h, 0.0)
        dinv_ref[...] = dinv

    row_spec = pl.BlockSpec((BLK, D), lambda i: (i, 0))
    col_spec = pl.BlockSpec((BLK, 1), lambda i: (i, 0))
    return pl.pallas_call(
        body,
        grid=(GRID,),
        in_specs=[row_spec, row_spec, col_spec, col_spec,
                  pl.BlockSpec((D, D), lambda i: (0, 0)),
                  pl.BlockSpec((1, D), lambda i: (0, 0))],
        out_specs=[row_spec, col_spec],
        out_shape=[jax.ShapeDtypeStruct((N_PAD, D), jnp.float32),
                   jax.ShapeDtypeStruct((N_PAD, 1), jnp.float32)],
    )(agg0, agg1, deg0, deg1, W, b2d)


def _tc_layer2(agg0, agg1, dinv, W, b2d):
    def body(a0, a1, dv, w, b, o_ref):
        a = (a0[...] + a1[...]) * dv[...]
        o_ref[...] = jnp.dot(a, w[...],
                             preferred_element_type=jnp.float32) + b[...]

    row_spec = pl.BlockSpec((BLK, D), lambda i: (i, 0))
    col_spec = pl.BlockSpec((BLK, 1), lambda i: (i, 0))
    return pl.pallas_call(
        body,
        grid=(GRID,),
        in_specs=[row_spec, row_spec, col_spec,
                  pl.BlockSpec((D, D), lambda i: (0, 0)),
                  pl.BlockSpec((1, D), lambda i: (0, 0))],
        out_specs=row_spec,
        out_shape=jax.ShapeDtypeStruct((N_PAD, D), jnp.float32),
    )(agg0, agg1, dinv, W, b2d)


def kernel(x, edge_index, W1, b1, W2, b2):
    src = edge_index[0]
    dst = edge_index[1]

    pad = E_PAD - E
    src_pad = jnp.concatenate(
        [src, jnp.zeros((pad,), jnp.int32)]).reshape(N_CHUNKS, CHUNK)
    dst_pad = jnp.concatenate(
        [dst, jnp.full((pad,), N, jnp.int32)]).reshape(N_CHUNKS, CHUNK)

    zrows = jnp.zeros((ROWS_PER_TILE, D), jnp.float32)

    agg1p, degp = _sc_aggregate(x, src_pad, dst_pad, zrows, True)

    h, dinv = _tc_layer1(agg1p[0], agg1p[1],
                         degp[0].reshape(N_PAD, 1), degp[1].reshape(N_PAD, 1),
                         W1, b1.reshape(1, D))

    (agg2p,) = _sc_aggregate(h, src_pad, dst_pad, zrows, False)

    out = _tc_layer2(agg2p[0], agg2p[1], dinv, W2, b2.reshape(1, D))
    return out[:N]


# speedup vs baseline: 3.4028x; 1.0685x over previous
import functools

import jax
import jax.numpy as jnp
from jax import lax
from jax.experimental import pallas as pl
from jax.experimental.pallas import tpu as pltpu
from jax.experimental.pallas import tpu_sc as plsc

N = 10000
D = 128
E = 320000

NUM_CORES = 2
NUM_SUBCORES = 16
NUM_WORKERS = NUM_CORES * NUM_SUBCORES

CHUNK = 128
N_PAD = 10240
E_PAD = 327680
N_CHUNKS = E_PAD // CHUNK
CHUNKS_PER_W = N_CHUNKS // NUM_WORKERS
ROWS_PER_TILE = N_PAD // NUM_SUBCORES
LANES = 16


def _sc_aggregate(table, src2d, dst2d, zrows, with_deg):
    mesh = plsc.VectorSubcoreMesh(core_axis_name="c", subcore_axis_name="s")

    out_type = [jax.ShapeDtypeStruct((NUM_CORES, N_PAD, D), jnp.float32)]
    scratch = (
        [pltpu.VMEM((CHUNK,), jnp.int32) for _ in range(8)]
        + [pltpu.VMEM((CHUNK, D), jnp.float32) for _ in range(2)]
        + [pltpu.VMEM_SHARED((N_PAD, D), jnp.float32)]
        + [pltpu.SemaphoreType.DMA for _ in range(12)]
    )
    if with_deg:
        out_type.append(jax.ShapeDtypeStruct((NUM_CORES, N_PAD), jnp.float32))
        out_type.append(jax.ShapeDtypeStruct(
            (NUM_CORES, NUM_SUBCORES, N_PAD), jnp.float32))
        scratch += [
            pltpu.VMEM((N_PAD,), jnp.float32),
            pltpu.VMEM((ROWS_PER_TILE,), jnp.float32),
            pltpu.VMEM((ROWS_PER_TILE,), jnp.float32),
        ]

    NSLOT = 4
    NSTEP = CHUNKS_PER_W // NSLOT

    @functools.partial(
        pl.kernel, mesh=mesh,
        compiler_params=pltpu.CompilerParams(needs_layout_passes=False),
        out_type=out_type, scratch_types=scratch)
    def k(table_hbm, src_hbm, dst_hbm, zrows_hbm, agg_out, *rest):
        if with_deg:
            deg_out, stage = rest[0], rest[1]
            rest = rest[2:]
            deg_v, acc_v, tmp_v = rest[23:]
        isl = rest[0:4]
        dsl = rest[4:8]
        rws = rest[8:10]
        agg_sh = rest[10]
        isem = rest[11:15]
        dsem = rest[15:19]
        gsem = rest[19:21]
        ssem = rest[21:23]

        cid = lax.axis_index("c")
        sid = lax.axis_index("s")
        wid = cid * NUM_SUBCORES + sid
        row0 = sid * ROWS_PER_TILE

        pltpu.sync_copy(zrows_hbm, agg_sh.at[pl.ds(row0, ROWS_PER_TILE)])
        if with_deg:
            def zb(j, c):
                deg_v[pl.ds(j * LANES, LANES)] = jnp.zeros((LANES,),
                                                           jnp.float32)
                return c
            lax.fori_loop(0, N_PAD // LANES, zb, 0)
        plsc.subcore_barrier()

        chunk0 = wid * CHUNKS_PER_W

        def hist(idx_ref):
            def hb(j, c2):
                iv = idx_ref[pl.ds(j * LANES, LANES)]
                plsc.addupdate_scatter(
                    deg_v, [iv], jnp.ones((LANES,), jnp.float32))
                return c2
            lax.fori_loop(0, CHUNK // LANES, hb, 0)

        for s in range(NSLOT):
            pltpu.async_copy(src_hbm.at[chunk0 + s], isl[s], isem[s])
            pltpu.async_copy(dst_hbm.at[chunk0 + s], dsl[s], dsem[s])

        def step(g, carry):
            for b in range(NSLOT):
                r = b % 2
                c = chunk0 + g * NSLOT + b

                def drain_and_refill():
                    pltpu.make_async_copy(
                        table_hbm.at[pl.ds(0, CHUNK)], rws[r],
                        ssem[r]).wait()
                    s2 = (b + 2) % NSLOT
                    pltpu.async_copy(src_hbm.at[c + 2], isl[s2], isem[s2])
                    pltpu.async_copy(dst_hbm.at[c + 2], dsl[s2], dsem[s2])
                if b < 2:
                    @pl.when(g > 0)
                    def _():
                        drain_and_refill()
                else:
                    @pl.when(g < NSTEP - 1)
                    def _():
                        drain_and_refill()
                    @pl.when(g == NSTEP - 1)
                    def _():
                        pltpu.make_async_copy(
                            table_hbm.at[pl.ds(0, CHUNK)], rws[r],
                            ssem[r]).wait()

                pltpu.make_async_copy(src_hbm.at[c], isl[b], isem[b]).wait()
                pltpu.make_async_copy(dst_hbm.at[c], dsl[b], dsem[b]).wait()
                gh = pltpu.async_copy(table_hbm.at[isl[b]], rws[r], gsem[r])
                if with_deg:
                    hist(dsl[b])
                gh.wait()
                pltpu.async_copy(rws[r], agg_sh.at[dsl[b]], ssem[r],
                                 add=True)
            return carry

        lax.fori_loop(0, NSTEP, step, 0)

        for r in range(2):
            pltpu.make_async_copy(table_hbm.at[pl.ds(0, CHUNK)], rws[r],
                                  ssem[r]).wait()

        if with_deg:
            pltpu.sync_copy(deg_v, stage.at[cid, sid])
        plsc.subcore_barrier()

        pltpu.sync_copy(agg_sh.at[pl.ds(row0, ROWS_PER_TILE)],
                        agg_out.at[cid, pl.ds(row0, ROWS_PER_TILE)])

        if with_deg:
            def zb2(j, c):
                acc_v[pl.ds(j * LANES, LANES)] = jnp.zeros((LANES,),
                                                           jnp.float32)
                return c
            lax.fori_loop(0, ROWS_PER_TILE // LANES, zb2, 0)

            def rb(t, c):
                pltpu.sync_copy(stage.at[cid, t, pl.ds(row0, ROWS_PER_TILE)],
                                tmp_v)

                def ab(j, c2):
                    s = pl.ds(j * LANES, LANES)
                    acc_v[s] = acc_v[s] + tmp_v[s]
                    return c2
                lax.fori_loop(0, ROWS_PER_TILE // LANES, ab, 0)
                return c
            lax.fori_loop(0, NUM_SUBCORES, rb, 0)
            pltpu.sync_copy(acc_v, deg_out.at[cid, pl.ds(row0,
                                                         ROWS_PER_TILE)])

    res = k(table, src2d, dst2d, zrows)
    if not isinstance(res, (list, tuple)):
        res = (res,)
    return res


BLK = 1024
GRID = N_PAD // BLK


def _tc_layer1(agg0, agg1, deg0, deg1, W, b2d):
    def body(a0, a1, d0, d1, w, b, h_ref, dinv_ref):
        deg = d0[...] + d1[...]
        dinv = 1.0 / jnp.maximum(deg, 1.0)
        a = (a0[...] + a1[...]) * dinv
        h = jnp.dot(a, w[...], preferred_element_type=jnp.float32) + b[...]
        h_ref[...] = jnp.maximum(h, 0.0)
        dinv_ref[...] = dinv

    row_spec = pl.BlockSpec((BLK, D), lambda i: (i, 0))
    col_spec = pl.BlockSpec((BLK, 1), lambda i: (i, 0))
    return pl.pallas_call(
        body,
        grid=(GRID,),
        in_specs=[row_spec, row_spec, col_spec, col_spec,
                  pl.BlockSpec((D, D), lambda i: (0, 0)),
                  pl.BlockSpec((1, D), lambda i: (0, 0))],
        out_specs=[row_spec, col_spec],
        out_shape=[jax.ShapeDtypeStruct((N_PAD, D), jnp.float32),
                   jax.ShapeDtypeStruct((N_PAD, 1), jnp.float32)],
    )(agg0, agg1, deg0, deg1, W, b2d)


def _tc_layer2(agg0, agg1, dinv, W, b2d):
    def body(a0, a1, dv, w, b, o_ref):
        a = (a0[...] + a1[...]) * dv[...]
        o_ref[...] = jnp.dot(a, w[...],
                             preferred_element_type=jnp.float32) + b[...]

    row_spec = pl.BlockSpec((BLK, D), lambda i: (i, 0))
    col_spec = pl.BlockSpec((BLK, 1), lambda i: (i, 0))
    return pl.pallas_call(
        body,
        grid=(GRID,),
        in_specs=[row_spec, row_spec, col_spec,
                  pl.BlockSpec((D, D), lambda i: (0, 0)),
                  pl.BlockSpec((1, D), lambda i: (0, 0))],
        out_specs=row_spec,
        out_shape=jax.ShapeDtypeStruct((N_PAD, D), jnp.float32),
    )(agg0, agg1, dinv, W, b2d)


def kernel(x, edge_index, W1, b1, W2, b2):
    src = edge_index[0]
    dst = edge_index[1]

    pad = E_PAD - E
    src_pad = jnp.concatenate(
        [src, jnp.zeros((pad,), jnp.int32)]).reshape(N_CHUNKS, CHUNK)
    dst_pad = jnp.concatenate(
        [dst, jnp.full((pad,), N, jnp.int32)]).reshape(N_CHUNKS, CHUNK)

    zrows = jnp.zeros((ROWS_PER_TILE, D), jnp.float32)

    agg1p, degp, _ = _sc_aggregate(x, src_pad, dst_pad, zrows, True)

    h, dinv = _tc_layer1(agg1p[0], agg1p[1],
                         degp[0].reshape(N_PAD, 1), degp[1].reshape(N_PAD, 1),
                         W1, b1.reshape(1, D))

    (agg2p,) = _sc_aggregate(h, src_pad, dst_pad, zrows, False)

    out = _tc_layer2(agg2p[0], agg2p[1], dinv, W2, b2.reshape(1, D))
    return out[:N]


# speedup vs baseline: 3.6770x; 1.0806x over previous
import functools

import jax
import jax.numpy as jnp
from jax import lax
from jax.experimental import pallas as pl
from jax.experimental.pallas import tpu as pltpu
from jax.experimental.pallas import tpu_sc as plsc

N = 10000
D = 128
E = 320000

NUM_CORES = 2
NUM_SUBCORES = 16
NUM_WORKERS = NUM_CORES * NUM_SUBCORES

CHUNK = 128
N_PAD = 10240
E_PAD = 327680
N_CHUNKS = E_PAD // CHUNK
ROWS_PER_TILE = N_PAD // NUM_SUBCORES
LANES = 16

CH0 = 1792
CH1 = N_CHUNKS - CH0
CPW0 = CH0 // NUM_SUBCORES
CPW1 = CH1 // NUM_SUBCORES


def _sc_aggregate(table, src2d, dst2d, zrows, with_deg):
    mesh = plsc.VectorSubcoreMesh(core_axis_name="c", subcore_axis_name="s")

    out_type = [jax.ShapeDtypeStruct((NUM_CORES, N_PAD, D), jnp.float32)]
    scratch = (
        [pltpu.VMEM((CHUNK,), jnp.int32) for _ in range(8)]
        + [pltpu.VMEM((CHUNK, D), jnp.float32) for _ in range(2)]
        + [pltpu.VMEM_SHARED((N_PAD, D), jnp.float32)]
        + [pltpu.SemaphoreType.DMA for _ in range(12)]
    )
    if with_deg:
        out_type.append(jax.ShapeDtypeStruct((NUM_CORES, N_PAD), jnp.float32))
        out_type.append(jax.ShapeDtypeStruct(
            (NUM_CORES, NUM_SUBCORES, N_PAD), jnp.float32))
        scratch += [
            pltpu.VMEM((N_PAD,), jnp.float32),
            pltpu.VMEM((ROWS_PER_TILE,), jnp.float32),
            pltpu.VMEM((ROWS_PER_TILE,), jnp.float32),
        ]

    NSLOT = 4

    @functools.partial(
        pl.kernel, mesh=mesh,
        compiler_params=pltpu.CompilerParams(needs_layout_passes=False),
        out_type=out_type, scratch_types=scratch)
    def k(table_hbm, src_hbm, dst_hbm, zrows_hbm, agg_out, *rest):
        if with_deg:
            deg_out, stage = rest[0], rest[1]
            rest = rest[2:]
            deg_v, acc_v, tmp_v = rest[23:]
        isl = rest[0:4]
        dsl = rest[4:8]
        rws = rest[8:10]
        agg_sh = rest[10]
        isem = rest[11:15]
        dsem = rest[15:19]
        gsem = rest[19:21]
        ssem = rest[21:23]

        cid = lax.axis_index("c")
        sid = lax.axis_index("s")
        wid = cid * NUM_SUBCORES + sid
        row0 = sid * ROWS_PER_TILE

        pltpu.sync_copy(zrows_hbm, agg_sh.at[pl.ds(row0, ROWS_PER_TILE)])
        if with_deg:
            def zb(j, c):
                deg_v[pl.ds(j * LANES, LANES)] = jnp.zeros((LANES,),
                                                           jnp.float32)
                return c
            lax.fori_loop(0, N_PAD // LANES, zb, 0)
        plsc.subcore_barrier()

        chunk0 = jnp.where(cid == 0, sid * CPW0, CH0 + sid * CPW1)
        nstep = jnp.where(cid == 0, CPW0 // NSLOT, CPW1 // NSLOT)

        def hist(idx_ref):
            def hb(j, c2):
                iv = idx_ref[pl.ds(j * LANES, LANES)]
                plsc.addupdate_scatter(
                    deg_v, [iv], jnp.ones((LANES,), jnp.float32))
                return c2
            lax.fori_loop(0, CHUNK // LANES, hb, 0)

        for s in range(NSLOT):
            pltpu.async_copy(src_hbm.at[chunk0 + s], isl[s], isem[s])
            pltpu.async_copy(dst_hbm.at[chunk0 + s], dsl[s], dsem[s])

        def step(g, carry):
            for b in range(NSLOT):
                r = b % 2
                c = chunk0 + g * NSLOT + b

                def drain_and_refill():
                    pltpu.make_async_copy(
                        table_hbm.at[pl.ds(0, CHUNK)], rws[r],
                        ssem[r]).wait()
                    s2 = (b + 2) % NSLOT
                    pltpu.async_copy(src_hbm.at[c + 2], isl[s2], isem[s2])
                    pltpu.async_copy(dst_hbm.at[c + 2], dsl[s2], dsem[s2])
                if b < 2:
                    @pl.when(g > 0)
                    def _():
                        drain_and_refill()
                else:
                    @pl.when(g < nstep - 1)
                    def _():
                        drain_and_refill()
                    @pl.when(g == nstep - 1)
                    def _():
                        pltpu.make_async_copy(
                            table_hbm.at[pl.ds(0, CHUNK)], rws[r],
                            ssem[r]).wait()

                pltpu.make_async_copy(src_hbm.at[c], isl[b], isem[b]).wait()
                pltpu.make_async_copy(dst_hbm.at[c], dsl[b], dsem[b]).wait()
                gh = pltpu.async_copy(table_hbm.at[isl[b]], rws[r], gsem[r])
                if with_deg:
                    hist(dsl[b])
                gh.wait()
                pltpu.async_copy(rws[r], agg_sh.at[dsl[b]], ssem[r],
                                 add=True)
            return carry

        lax.fori_loop(0, nstep, step, 0)

        for r in range(2):
            pltpu.make_async_copy(table_hbm.at[pl.ds(0, CHUNK)], rws[r],
                                  ssem[r]).wait()

        if with_deg:
            pltpu.sync_copy(deg_v, stage.at[cid, sid])
        plsc.subcore_barrier()

        pltpu.sync_copy(agg_sh.at[pl.ds(row0, ROWS_PER_TILE)],
                        agg_out.at[cid, pl.ds(row0, ROWS_PER_TILE)])

        if with_deg:
            def zb2(j, c):
                acc_v[pl.ds(j * LANES, LANES)] = jnp.zeros((LANES,),
                                                           jnp.float32)
                return c
            lax.fori_loop(0, ROWS_PER_TILE // LANES, zb2, 0)

            def rb(t, c):
                pltpu.sync_copy(stage.at[cid, t, pl.ds(row0, ROWS_PER_TILE)],
                                tmp_v)

                def ab(j, c2):
                    s = pl.ds(j * LANES, LANES)
                    acc_v[s] = acc_v[s] + tmp_v[s]
                    return c2
                lax.fori_loop(0, ROWS_PER_TILE // LANES, ab, 0)
                return c
            lax.fori_loop(0, NUM_SUBCORES, rb, 0)
            pltpu.sync_copy(acc_v, deg_out.at[cid, pl.ds(row0,
                                                         ROWS_PER_TILE)])

    res = k(table, src2d, dst2d, zrows)
    if not isinstance(res, (list, tuple)):
        res = (res,)
    return res


BLK = 1024
GRID = N_PAD // BLK


def _tc_layer1(agg0, agg1, deg0, deg1, W, b2d):
    def body(a0, a1, d0, d1, w, b, h_ref, dinv_ref):
        deg = d0[...] + d1[...]
        dinv = 1.0 / jnp.maximum(deg, 1.0)
        a = (a0[...] + a1[...]) * dinv
        h = jnp.dot(a, w[...], preferred_element_type=jnp.float32) + b[...]
        h_ref[...] = jnp.maximum(h, 0.0)
        dinv_ref[...] = dinv

    row_spec = pl.BlockSpec((BLK, D), lambda i: (i, 0))
    col_spec = pl.BlockSpec((BLK, 1), lambda i: (i, 0))
    return pl.pallas_call(
        body,
        grid=(GRID,),
        in_specs=[row_spec, row_spec, col_spec, col_spec,
                  pl.BlockSpec((D, D), lambda i: (0, 0)),
                  pl.BlockSpec((1, D), lambda i: (0, 0))],
        out_specs=[row_spec, col_spec],
        out_shape=[jax.ShapeDtypeStruct((N_PAD, D), jnp.float32),
                   jax.ShapeDtypeStruct((N_PAD, 1), jnp.float32)],
    )(agg0, agg1, deg0, deg1, W, b2d)


def _tc_layer2(agg0, agg1, dinv, W, b2d):
    def body(a0, a1, dv, w, b, o_ref):
        a = (a0[...] + a1[...]) * dv[...]
        o_ref[...] = jnp.dot(a, w[...],
                             preferred_element_type=jnp.float32) + b[...]

    row_spec = pl.BlockSpec((BLK, D), lambda i: (i, 0))
    col_spec = pl.BlockSpec((BLK, 1), lambda i: (i, 0))
    return pl.pallas_call(
        body,
        grid=(GRID,),
        in_specs=[row_spec, row_spec, col_spec,
                  pl.BlockSpec((D, D), lambda i: (0, 0)),
                  pl.BlockSpec((1, D), lambda i: (0, 0))],
        out_specs=row_spec,
        out_shape=jax.ShapeDtypeStruct((N_PAD, D), jnp.float32),
    )(agg0, agg1, dinv, W, b2d)


def kernel(x, edge_index, W1, b1, W2, b2):
    src = edge_index[0]
    dst = edge_index[1]

    pad = E_PAD - E
    src_pad = jnp.concatenate(
        [src, jnp.zeros((pad,), jnp.int32)]).reshape(N_CHUNKS, CHUNK)
    dst_pad = jnp.concatenate(
        [dst, jnp.full((pad,), N, jnp.int32)]).reshape(N_CHUNKS, CHUNK)

    zrows = jnp.zeros((ROWS_PER_TILE, D), jnp.float32)

    agg1p, degp, _ = _sc_aggregate(x, src_pad, dst_pad, zrows, True)

    h, dinv = _tc_layer1(agg1p[0], agg1p[1],
                         degp[0].reshape(N_PAD, 1), degp[1].reshape(N_PAD, 1),
                         W1, b1.reshape(1, D))

    (agg2p,) = _sc_aggregate(h, src_pad, dst_pad, zrows, False)

    out = _tc_layer2(agg2p[0], agg2p[1], dinv, W2, b2.reshape(1, D))
    return out[:N]


# speedup vs baseline: 3.7098x; 1.0089x over previous
import functools

import jax
import jax.numpy as jnp
from jax import lax
from jax.experimental import pallas as pl
from jax.experimental.pallas import tpu as pltpu
from jax.experimental.pallas import tpu_sc as plsc

N = 10000
D = 128
E = 320000

NUM_CORES = 2
NUM_SUBCORES = 16
NUM_WORKERS = NUM_CORES * NUM_SUBCORES

CHUNK = 128
N_PAD = 10240
E_PAD = 327680
N_CHUNKS = E_PAD // CHUNK
ROWS_PER_TILE = N_PAD // NUM_SUBCORES
LANES = 16

CH0 = 1792
CH1 = N_CHUNKS - CH0
CPW0 = CH0 // NUM_SUBCORES
CPW1 = CH1 // NUM_SUBCORES


def _sc_aggregate(table, src2d, dst2d, zrows, with_deg):
    mesh = plsc.VectorSubcoreMesh(core_axis_name="c", subcore_axis_name="s")

    out_type = [jax.ShapeDtypeStruct((NUM_CORES, N_PAD, D), jnp.float32)]
    scratch = (
        [pltpu.VMEM((CHUNK,), jnp.int32) for _ in range(8)]
        + [pltpu.VMEM((CHUNK, D), jnp.float32) for _ in range(2)]
        + [pltpu.VMEM_SHARED((N_PAD, D), jnp.float32)]
        + [pltpu.SemaphoreType.DMA for _ in range(12)]
    )
    if with_deg:
        out_type.append(jax.ShapeDtypeStruct((NUM_CORES, N_PAD), jnp.float32))
        out_type.append(jax.ShapeDtypeStruct(
            (NUM_CORES, NUM_SUBCORES, N_PAD), jnp.float32))
        scratch += [
            pltpu.VMEM((N_PAD,), jnp.float32),
            pltpu.VMEM((ROWS_PER_TILE,), jnp.float32),
            pltpu.VMEM((ROWS_PER_TILE,), jnp.float32),
        ]

    NSLOT = 4

    @functools.partial(
        pl.kernel, mesh=mesh,
        compiler_params=pltpu.CompilerParams(needs_layout_passes=False),
        out_type=out_type, scratch_types=scratch)
    def k(table_hbm, src_hbm, dst_hbm, zrows_hbm, agg_out, *rest):
        if with_deg:
            deg_out, stage = rest[0], rest[1]
            rest = rest[2:]
            deg_v, acc_v, tmp_v = rest[23:]
        isl = rest[0:4]
        dsl = rest[4:8]
        rws = rest[8:10]
        agg_sh = rest[10]
        isem = rest[11:15]
        dsem = rest[15:19]
        gsem = rest[19:21]
        ssem = rest[21:23]

        cid = lax.axis_index("c")
        sid = lax.axis_index("s")
        wid = cid * NUM_SUBCORES + sid
        row0 = sid * ROWS_PER_TILE

        pltpu.sync_copy(zrows_hbm, agg_sh.at[pl.ds(row0, ROWS_PER_TILE)])
        if with_deg:
            def zb(j, c):
                deg_v[pl.ds(j * LANES, LANES)] = jnp.zeros((LANES,),
                                                           jnp.float32)
                return c
            lax.fori_loop(0, N_PAD // LANES, zb, 0)
        plsc.subcore_barrier()

        chunk0 = jnp.where(cid == 0, sid * CPW0, CH0 + sid * CPW1)
        nstep = jnp.where(cid == 0, CPW0 // NSLOT, CPW1 // NSLOT)

        def hist(idx_ref):
            def hb(j, c2):
                iv = idx_ref[pl.ds(j * LANES, LANES)]
                plsc.addupdate_scatter(
                    deg_v, [iv], jnp.ones((LANES,), jnp.float32))
                return c2
            lax.fori_loop(0, CHUNK // LANES, hb, 0)

        for s in range(NSLOT):
            pltpu.async_copy(src_hbm.at[chunk0 + s], isl[s], isem[s])
            pltpu.async_copy(dst_hbm.at[chunk0 + s], dsl[s], dsem[s])

        def step(g, carry):
            for b in range(NSLOT):
                r = b % 2
                c = chunk0 + g * NSLOT + b

                def drain():
                    pltpu.make_async_copy(
                        table_hbm.at[pl.ds(0, CHUNK)], rws[r],
                        ssem[r]).wait()

                def refill():
                    s2 = (b + 2) % NSLOT
                    pltpu.async_copy(src_hbm.at[c + 2], isl[s2], isem[s2])
                    pltpu.async_copy(dst_hbm.at[c + 2], dsl[s2], dsem[s2])

                if b < 2:
                    @pl.when(g > 0)
                    def _():
                        drain()
                        refill()
                else:
                    drain()
                    @pl.when(g < nstep - 1)
                    def _():
                        refill()

                pltpu.make_async_copy(src_hbm.at[c], isl[b], isem[b]).wait()
                pltpu.make_async_copy(dst_hbm.at[c], dsl[b], dsem[b]).wait()
                pltpu.async_copy(table_hbm.at[isl[b]], rws[r], gsem[r])

                rp = (b + 1) % 2
                bp = (b - 1) % NSLOT

                def consume_prev():
                    pltpu.make_async_copy(
                        table_hbm.at[pl.ds(0, CHUNK)], rws[rp],
                        gsem[rp]).wait()
                    if with_deg:
                        hist(dsl[bp])
                    pltpu.async_copy(rws[rp], agg_sh.at[dsl[bp]], ssem[rp],
                                     add=True)
                if b == 0:
                    @pl.when(g > 0)
                    def _():
                        consume_prev()
                else:
                    consume_prev()
            return carry

        lax.fori_loop(0, nstep, step, 0)

        pltpu.make_async_copy(table_hbm.at[pl.ds(0, CHUNK)], rws[1],
                              gsem[1]).wait()
        if with_deg:
            hist(dsl[3])
        pltpu.async_copy(rws[1], agg_sh.at[dsl[3]], ssem[1], add=True)
        for r in range(2):
            pltpu.make_async_copy(table_hbm.at[pl.ds(0, CHUNK)], rws[r],
                                  ssem[r]).wait()

        if with_deg:
            pltpu.sync_copy(deg_v, stage.at[cid, sid])
        plsc.subcore_barrier()

        pltpu.sync_copy(agg_sh.at[pl.ds(row0, ROWS_PER_TILE)],
                        agg_out.at[cid, pl.ds(row0, ROWS_PER_TILE)])

        if with_deg:
            def zb2(j, c):
                acc_v[pl.ds(j * LANES, LANES)] = jnp.zeros((LANES,),
                                                           jnp.float32)
                return c
            lax.fori_loop(0, ROWS_PER_TILE // LANES, zb2, 0)

            def rb(t, c):
                pltpu.sync_copy(stage.at[cid, t, pl.ds(row0, ROWS_PER_TILE)],
                                tmp_v)

                def ab(j, c2):
                    s = pl.ds(j * LANES, LANES)
                    acc_v[s] = acc_v[s] + tmp_v[s]
                    return c2
                lax.fori_loop(0, ROWS_PER_TILE // LANES, ab, 0)
                return c
            lax.fori_loop(0, NUM_SUBCORES, rb, 0)
            pltpu.sync_copy(acc_v, deg_out.at[cid, pl.ds(row0,
                                                         ROWS_PER_TILE)])

    res = k(table, src2d, dst2d, zrows)
    if not isinstance(res, (list, tuple)):
        res = (res,)
    return res


BLK = 1024
GRID = N_PAD // BLK


def _tc_layer1(agg0, agg1, deg0, deg1, W, b2d):
    def body(a0, a1, d0, d1, w, b, h_ref, dinv_ref):
        deg = d0[...] + d1[...]
        dinv = 1.0 / jnp.maximum(deg, 1.0)
        a = (a0[...] + a1[...]) * dinv
        h = jnp.dot(a, w[...], preferred_element_type=jnp.float32) + b[...]
        h_ref[...] = jnp.maximum(h, 0.0)
        dinv_ref[...] = dinv

    row_spec = pl.BlockSpec((BLK, D), lambda i: (i, 0))
    col_spec = pl.BlockSpec((BLK, 1), lambda i: (i, 0))
    return pl.pallas_call(
        body,
        grid=(GRID,),
        in_specs=[row_spec, row_spec, col_spec, col_spec,
                  pl.BlockSpec((D, D), lambda i: (0, 0)),
                  pl.BlockSpec((1, D), lambda i: (0, 0))],
        out_specs=[row_spec, col_spec],
        out_shape=[jax.ShapeDtypeStruct((N_PAD, D), jnp.float32),
                   jax.ShapeDtypeStruct((N_PAD, 1), jnp.float32)],
    )(agg0, agg1, deg0, deg1, W, b2d)


def _tc_layer2(agg0, agg1, dinv, W, b2d):
    def body(a0, a1, dv, w, b, o_ref):
        a = (a0[...] + a1[...]) * dv[...]
        o_ref[...] = jnp.dot(a, w[...],
                             preferred_element_type=jnp.float32) + b[...]

    row_spec = pl.BlockSpec((BLK, D), lambda i: (i, 0))
    col_spec = pl.BlockSpec((BLK, 1), lambda i: (i, 0))
    return pl.pallas_call(
        body,
        grid=(GRID,),
        in_specs=[row_spec, row_spec, col_spec,
                  pl.BlockSpec((D, D), lambda i: (0, 0)),
                  pl.BlockSpec((1, D), lambda i: (0, 0))],
        out_specs=row_spec,
        out_shape=jax.ShapeDtypeStruct((N_PAD, D), jnp.float32),
    )(agg0, agg1, dinv, W, b2d)


def kernel(x, edge_index, W1, b1, W2, b2):
    src = edge_index[0]
    dst = edge_index[1]

    pad = E_PAD - E
    src_pad = jnp.concatenate(
        [src, jnp.zeros((pad,), jnp.int32)]).reshape(N_CHUNKS, CHUNK)
    dst_pad = jnp.concatenate(
        [dst, jnp.full((pad,), N, jnp.int32)]).reshape(N_CHUNKS, CHUNK)

    zrows = jnp.zeros((ROWS_PER_TILE, D), jnp.float32)

    agg1p, degp, _ = _sc_aggregate(x, src_pad, dst_pad, zrows, True)

    h, dinv = _tc_layer1(agg1p[0], agg1p[1],
                         degp[0].reshape(N_PAD, 1), degp[1].reshape(N_PAD, 1),
                         W1, b1.reshape(1, D))

    (agg2p,) = _sc_aggregate(h, src_pad, dst_pad, zrows, False)

    out = _tc_layer2(agg2p[0], agg2p[1], dinv, W2, b2.reshape(1, D))
    return out[:N]


# speedup vs baseline: 4.1370x; 1.1152x over previous
import functools

import jax
import jax.numpy as jnp
from jax import lax
from jax.experimental import pallas as pl
from jax.experimental.pallas import tpu as pltpu
from jax.experimental.pallas import tpu_sc as plsc

N = 10000
D = 128
E = 320000

NUM_CORES = 2
NUM_SUBCORES = 16
NUM_WORKERS = NUM_CORES * NUM_SUBCORES

CHUNK = 128
N_PAD = 10240
E_PAD = 327680
N_CHUNKS = E_PAD // CHUNK
ROWS_PER_TILE = N_PAD // NUM_SUBCORES
LANES = 16

CH0 = 1792
CH1 = N_CHUNKS - CH0
CPW0 = CH0 // NUM_SUBCORES
CPW1 = CH1 // NUM_SUBCORES


def _sc_aggregate(table, src2d, dst2d, with_deg):
    mesh = plsc.VectorSubcoreMesh(core_axis_name="c", subcore_axis_name="s")

    out_type = [jax.ShapeDtypeStruct((NUM_CORES, N_PAD, D), jnp.float32)]
    scratch = (
        [pltpu.VMEM((CHUNK,), jnp.int32) for _ in range(8)]
        + [pltpu.VMEM((CHUNK, D), jnp.float32) for _ in range(2)]
        + [pltpu.VMEM_SHARED((N_PAD, D), jnp.float32)]
        + [pltpu.SemaphoreType.DMA for _ in range(12)]
    )
    if with_deg:
        out_type.append(jax.ShapeDtypeStruct((NUM_CORES, N_PAD), jnp.float32))
        out_type.append(jax.ShapeDtypeStruct(
            (NUM_CORES, NUM_SUBCORES, N_PAD), jnp.float32))
        scratch += [
            pltpu.VMEM((N_PAD,), jnp.float32),
            pltpu.VMEM((ROWS_PER_TILE,), jnp.float32),
            pltpu.VMEM((ROWS_PER_TILE,), jnp.float32),
        ]

    NSLOT = 4

    @functools.partial(
        pl.kernel, mesh=mesh,
        compiler_params=pltpu.CompilerParams(needs_layout_passes=False),
        out_type=out_type, scratch_types=scratch)
    def k(table_hbm, src_hbm, dst_hbm, agg_out, *rest):
        if with_deg:
            deg_out, stage = rest[0], rest[1]
            rest = rest[2:]
            deg_v, acc_v, tmp_v = rest[23:]
        isl = rest[0:4]
        dsl = rest[4:8]
        rws = rest[8:10]
        agg_sh = rest[10]
        isem = rest[11:15]
        dsem = rest[15:19]
        gsem = rest[19:21]
        ssem = rest[21:23]

        cid = lax.axis_index("c")
        sid = lax.axis_index("s")
        wid = cid * NUM_SUBCORES + sid
        row0 = sid * ROWS_PER_TILE

        def zrow(i, c):
            def zcol(j, c2):
                rws[0][i, pl.ds(j * LANES, LANES)] = jnp.zeros((LANES,),
                                                               jnp.float32)
                return c2
            lax.fori_loop(0, D // LANES, zcol, 0)
            return c
        lax.fori_loop(0, CHUNK, zrow, 0)

        def zstripe(s, c):
            pltpu.sync_copy(rws[0],
                            agg_sh.at[pl.ds(row0 + s * CHUNK, CHUNK)])
            return c
        lax.fori_loop(0, ROWS_PER_TILE // CHUNK, zstripe, 0)
        if with_deg:
            def zb(j, c):
                deg_v[pl.ds(j * LANES, LANES)] = jnp.zeros((LANES,),
                                                           jnp.float32)
                return c
            lax.fori_loop(0, N_PAD // LANES, zb, 0)
        plsc.subcore_barrier()

        chunk0 = jnp.where(cid == 0, sid * CPW0, CH0 + sid * CPW1)
        nstep = jnp.where(cid == 0, CPW0 // NSLOT, CPW1 // NSLOT)

        def hist(idx_ref):
            def hb(j, c2):
                iv = idx_ref[pl.ds(j * LANES, LANES)]
                plsc.addupdate_scatter(
                    deg_v, [iv], jnp.ones((LANES,), jnp.float32))
                return c2
            lax.fori_loop(0, CHUNK // LANES, hb, 0)

        for s in range(NSLOT):
            pltpu.async_copy(src_hbm.at[chunk0 + s], isl[s], isem[s])
            pltpu.async_copy(dst_hbm.at[chunk0 + s], dsl[s], dsem[s])

        def step(g, carry):
            for b in range(NSLOT):
                r = b % 2
                c = chunk0 + g * NSLOT + b

                def drain():
                    pltpu.make_async_copy(
                        table_hbm.at[pl.ds(0, CHUNK)], rws[r],
                        ssem[r]).wait()

                def refill():
                    s2 = (b + 2) % NSLOT
                    pltpu.async_copy(src_hbm.at[c + 2], isl[s2], isem[s2])
                    pltpu.async_copy(dst_hbm.at[c + 2], dsl[s2], dsem[s2])

                if b < 2:
                    @pl.when(g > 0)
                    def _():
                        drain()
                        refill()
                else:
                    drain()
                    @pl.when(g < nstep - 1)
                    def _():
                        refill()

                pltpu.make_async_copy(src_hbm.at[c], isl[b], isem[b]).wait()
                pltpu.make_async_copy(dst_hbm.at[c], dsl[b], dsem[b]).wait()
                pltpu.async_copy(table_hbm.at[isl[b]], rws[r], gsem[r])

                rp = (b + 1) % 2
                bp = (b - 1) % NSLOT

                def consume_prev():
                    pltpu.make_async_copy(
                        table_hbm.at[pl.ds(0, CHUNK)], rws[rp],
                        gsem[rp]).wait()
                    if with_deg:
                        hist(dsl[bp])
                    pltpu.async_copy(rws[rp], agg_sh.at[dsl[bp]], ssem[rp],
                                     add=True)
                if b == 0:
                    @pl.when(g > 0)
                    def _():
                        consume_prev()
                else:
                    consume_prev()
            return carry

        lax.fori_loop(0, nstep, step, 0)

        pltpu.make_async_copy(table_hbm.at[pl.ds(0, CHUNK)], rws[1],
                              gsem[1]).wait()
        if with_deg:
            hist(dsl[3])
        pltpu.async_copy(rws[1], agg_sh.at[dsl[3]], ssem[1], add=True)
        for r in range(2):
            pltpu.make_async_copy(table_hbm.at[pl.ds(0, CHUNK)], rws[r],
                                  ssem[r]).wait()

        if with_deg:
            pltpu.sync_copy(deg_v, stage.at[cid, sid])
        plsc.subcore_barrier()

        pltpu.sync_copy(agg_sh.at[pl.ds(row0, ROWS_PER_TILE)],
                        agg_out.at[cid, pl.ds(row0, ROWS_PER_TILE)])

        if with_deg:
            def zb2(j, c):
                acc_v[pl.ds(j * LANES, LANES)] = jnp.zeros((LANES,),
                                                           jnp.float32)
                return c
            lax.fori_loop(0, ROWS_PER_TILE // LANES, zb2, 0)

            def rb(t, c):
                pltpu.sync_copy(stage.at[cid, t, pl.ds(row0, ROWS_PER_TILE)],
                                tmp_v)

                def ab(j, c2):
                    s = pl.ds(j * LANES, LANES)
                    acc_v[s] = acc_v[s] + tmp_v[s]
                    return c2
                lax.fori_loop(0, ROWS_PER_TILE // LANES, ab, 0)
                return c
            lax.fori_loop(0, NUM_SUBCORES, rb, 0)
            pltpu.sync_copy(acc_v, deg_out.at[cid, pl.ds(row0,
                                                         ROWS_PER_TILE)])

    res = k(table, src2d, dst2d)
    if not isinstance(res, (list, tuple)):
        res = (res,)
    return res


BLK = 1024
GRID = N_PAD // BLK


def _tc_layer1(agg0, agg1, deg0, deg1, W, b2d):
    def body(a0, a1, d0, d1, w, b, h_ref, dinv_ref):
        deg = d0[...] + d1[...]
        dinv = 1.0 / jnp.maximum(deg, 1.0)
        a = (a0[...] + a1[...]) * dinv
        h = jnp.dot(a, w[...], preferred_element_type=jnp.float32) + b[...]
        h_ref[...] = jnp.maximum(h, 0.0)
        dinv_ref[...] = dinv

    row_spec = pl.BlockSpec((BLK, D), lambda i: (i, 0))
    col_spec = pl.BlockSpec((BLK, 1), lambda i: (i, 0))
    return pl.pallas_call(
        body,
        grid=(GRID,),
        in_specs=[row_spec, row_spec, col_spec, col_spec,
                  pl.BlockSpec((D, D), lambda i: (0, 0)),
                  pl.BlockSpec((1, D), lambda i: (0, 0))],
        out_specs=[row_spec, col_spec],
        out_shape=[jax.ShapeDtypeStruct((N_PAD, D), jnp.float32),
                   jax.ShapeDtypeStruct((N_PAD, 1), jnp.float32)],
    )(agg0, agg1, deg0, deg1, W, b2d)


def _tc_layer2(agg0, agg1, dinv, W, b2d):
    def body(a0, a1, dv, w, b, o_ref):
        a = (a0[...] + a1[...]) * dv[...]
        o_ref[...] = jnp.dot(a, w[...],
                             preferred_element_type=jnp.float32) + b[...]

    row_spec = pl.BlockSpec((BLK, D), lambda i: (i, 0))
    col_spec = pl.BlockSpec((BLK, 1), lambda i: (i, 0))
    return pl.pallas_call(
        body,
        grid=(GRID,),
        in_specs=[row_spec, row_spec, col_spec,
                  pl.BlockSpec((D, D), lambda i: (0, 0)),
                  pl.BlockSpec((1, D), lambda i: (0, 0))],
        out_specs=row_spec,
        out_shape=jax.ShapeDtypeStruct((N_PAD, D), jnp.float32),
    )(agg0, agg1, dinv, W, b2d)


def kernel(x, edge_index, W1, b1, W2, b2):
    src = edge_index[0]
    dst = edge_index[1]

    pad = E_PAD - E
    src_pad = jnp.concatenate(
        [src, jnp.zeros((pad,), jnp.int32)]).reshape(N_CHUNKS, CHUNK)
    dst_pad = jnp.concatenate(
        [dst, jnp.full((pad,), N, jnp.int32)]).reshape(N_CHUNKS, CHUNK)

    agg1p, degp, _ = _sc_aggregate(x, src_pad, dst_pad, True)

    h, dinv = _tc_layer1(agg1p[0], agg1p[1],
                         degp[0].reshape(N_PAD, 1), degp[1].reshape(N_PAD, 1),
                         W1, b1.reshape(1, D))

    (agg2p,) = _sc_aggregate(h, src_pad, dst_pad, False)

    out = _tc_layer2(agg2p[0], agg2p[1], dinv, W2, b2.reshape(1, D))
    return out[:N]
